# baseline (speedup 1.0000x reference)
import functools

import jax
import jax.numpy as jnp
from jax import lax
from jax.experimental import pallas as pl
from jax.experimental.pallas import tpu as pltpu
from jax.experimental.pallas import tpu_sc as plsc

VOCAB = 100000
D_MODEL = 1024
NUM_CORES = 2
NUM_SUBCORES = 16
NUM_WORKERS = NUM_CORES * NUM_SUBCORES

CHUNK = 16
NBUF = 8


def _embed_body(n_rows, x_hbm, table_hbm, out_hbm, idx_v, gsems):
    b_per_w = n_rows // NUM_WORKERS
    n_chunks = b_per_w // CHUNK
    seq_len = x_hbm.shape[1]
    w_per_row = seq_len // b_per_w
    wid = lax.axis_index("s") * NUM_CORES + lax.axis_index("c")
    row = wid // w_per_row
    col = (wid % w_per_row) * b_per_w
    pltpu.sync_copy(x_hbm.at[row, pl.ds(col, b_per_w)], idx_v)

    def issue(ch, b):
        vec = idx_v[pl.ds(ch * CHUNK, CHUNK)]
        for j in range(CHUNK):
            pltpu.async_copy(
                table_hbm.at[pl.ds(vec[j], 1)],
                out_hbm.at[row, pl.ds(col + ch * CHUNK + j, 1)],
                gsems.at[b],
            )

    def drain(ch, b):
        pltpu.make_async_copy(
            table_hbm.at[pl.ds(0, CHUNK)],
            out_hbm.at[0, pl.ds(0, CHUNK)],
            gsems.at[b],
        ).wait()

    for b in range(NBUF):
        issue(b, b)

    @pl.loop(0, n_chunks - NBUF, step=NBUF)
    def _chunks(c0):
        for b in range(NBUF):
            drain(c0 + b, b)
            issue(c0 + b + NBUF, b)

    c0 = n_chunks - NBUF
    for b in range(NBUF):
        drain(c0 + b, b)


def kernel(x, table):
    B, T = x.shape
    n_rows = B * T

    mesh = plsc.VectorSubcoreMesh(
        core_axis_name="c", subcore_axis_name="s",
        num_cores=NUM_CORES, num_subcores=NUM_SUBCORES,
    )
    b_per_w = n_rows // NUM_WORKERS
    run = pl.kernel(
        functools.partial(_embed_body, n_rows),
        out_type=jax.ShapeDtypeStruct((B, T, D_MODEL), jnp.float32),
        mesh=mesh,
        scratch_types=[
            pltpu.VMEM((b_per_w,), jnp.int32),
            pltpu.SemaphoreType.DMA((NBUF,)),
        ],
    )
    return run(x.astype(jnp.int32), table)
